# baseline (speedup 1.0000x reference)
import jax
import jax.numpy as jnp
from jax.experimental import pallas as pl
from jax.experimental.pallas import tpu as pltpu

_EPS = 1e-6
_FP8_MAX = 448.0

_TOKENS = 8192
_HIDDEN = 4096
_BT1 = 128
_BT2 = 512
_NB1 = _TOKENS // _BT1
_NB2 = _TOKENS // _BT2


def _pass1_kernel(hs_ref, w_ref, normed_ref, pamax_ref):
    y = hs_ref[0] + hs_ref[1] + hs_ref[2] + hs_ref[3]
    var = jnp.mean(y * y, axis=-1, keepdims=True)
    inv = jax.lax.rsqrt(var + _EPS)
    normed = y * inv * w_ref[...]
    normed_ref[...] = normed.astype(normed_ref.dtype)
    pamax_ref[...] = jnp.broadcast_to(jnp.max(jnp.abs(normed)), (1, 128))


def _pass2_kernel(pa_ref, normed_ref, q_ref, scale_ref):
    amax = jnp.max(pa_ref[...])
    scale = jnp.maximum(amax, 1e-12) / _FP8_MAX
    scale_ref[0, 0] = scale
    normed = normed_ref[...].astype(jnp.float32)
    q_ref[...] = jnp.clip(normed / scale, -_FP8_MAX, _FP8_MAX)


def kernel(hidden_states, residual, weight):
    del residual
    w2d = weight.reshape(1, _HIDDEN)

    normed16, pamax = pl.pallas_call(
        _pass1_kernel,
        grid=(_NB1,),
        in_specs=[
            pl.BlockSpec((4, _BT1, _HIDDEN), lambda i: (0, i, 0)),
            pl.BlockSpec((1, _HIDDEN), lambda i: (0, 0)),
        ],
        out_specs=[
            pl.BlockSpec((_BT1, _HIDDEN), lambda i: (i, 0)),
            pl.BlockSpec((1, 128), lambda i: (0, i)),
        ],
        out_shape=[
            jax.ShapeDtypeStruct((_TOKENS, _HIDDEN), jnp.bfloat16),
            jax.ShapeDtypeStruct((1, _NB1 * 128), jnp.float32),
        ],
        compiler_params=pltpu.CompilerParams(
            dimension_semantics=("parallel",),
            vmem_limit_bytes=56 * 1024 * 1024,
        ),
        name="allreduce_norm_stats",
    )(hidden_states, w2d)

    q, scale = pl.pallas_call(
        _pass2_kernel,
        grid=(_NB2,),
        in_specs=[
            pl.BlockSpec((1, _NB1 * 128), lambda i: (0, 0)),
            pl.BlockSpec((_BT2, _HIDDEN), lambda i: (i, 0)),
        ],
        out_specs=[
            pl.BlockSpec((_BT2, _HIDDEN), lambda i: (i, 0)),
            pl.BlockSpec(memory_space=pltpu.SMEM),
        ],
        out_shape=[
            jax.ShapeDtypeStruct((_TOKENS, _HIDDEN), jnp.float32),
            jax.ShapeDtypeStruct((1, 1), jnp.float32),
        ],
        compiler_params=pltpu.CompilerParams(
            dimension_semantics=("parallel",),
            vmem_limit_bytes=56 * 1024 * 1024,
        ),
        name="quant_scale",
    )(pamax, normed16)

    return q, scale.reshape(())


# speedup vs baseline: 1.0510x; 1.0510x over previous
import jax
import jax.numpy as jnp
from jax.experimental import pallas as pl
from jax.experimental.pallas import tpu as pltpu

_EPS = 1e-6
_FP8_MAX = 448.0

_TOKENS = 8192
_HIDDEN = 4096

_BT0 = 128
_NB0 = _TOKENS // _BT0
_N_SPILL = 32
_N_RES = _NB0 - _N_SPILL
_SPILL_TOKENS = _N_SPILL * _BT0
_RES_TOKENS = _N_RES * _BT0

_BT1 = 128
_NQ1 = _RES_TOKENS // _BT1
_RES_Q_BLOCK0 = _SPILL_TOKENS // _BT1

_BT2 = 512
_NB2 = _SPILL_TOKENS // _BT2


def _fused_kernel(hs_ref, w_ref, spill_ref, pamax_ref, q_ref, scale_ref,
                  res_ref, pstat_ref):
    i = pl.program_id(0)

    @pl.when(i < _NB0)
    def _phase0():
        y = hs_ref[0] + hs_ref[1] + hs_ref[2] + hs_ref[3]
        var = jnp.mean(y * y, axis=-1, keepdims=True)
        inv = jax.lax.rsqrt(var + _EPS)
        normed = y * inv * w_ref[...]
        blk_amax = jnp.broadcast_to(jnp.max(jnp.abs(normed)), (1, 128))
        pstat_ref[pl.ds(jnp.minimum(i, _NB0 - 1), 1), :] = blk_amax
        pamax_ref[...] = blk_amax
        normed_bf = normed.astype(jnp.bfloat16)

        @pl.when(i < _N_SPILL)
        def _spill():
            spill_ref[...] = normed_bf

        @pl.when(i >= _N_SPILL)
        def _resident():
            base = jnp.maximum(i - _N_SPILL, 0) * _BT0
            res_ref[pl.ds(base, _BT0), :] = normed_bf

    @pl.when(i >= _NB0)
    def _phase1():
        s = jnp.maximum(i - _NB0, 0)
        scale = jnp.maximum(jnp.max(pstat_ref[...]), 1e-12) / _FP8_MAX
        scale_ref[0, 0] = scale
        normed = res_ref[pl.ds(s * _BT1, _BT1), :].astype(jnp.float32)
        q_ref[...] = jnp.clip(normed / scale, -_FP8_MAX, _FP8_MAX)


def _spill_quant_kernel(pa_ref, spill_ref, q_in_ref, q_ref):
    del q_in_ref
    scale = jnp.maximum(jnp.max(pa_ref[...]), 1e-12) / _FP8_MAX
    normed = spill_ref[...].astype(jnp.float32)
    q_ref[...] = jnp.clip(normed / scale, -_FP8_MAX, _FP8_MAX)


def kernel(hidden_states, residual, weight):
    del residual
    w2d = weight.reshape(1, _HIDDEN)

    spill, pamax, q_partial, scale = pl.pallas_call(
        _fused_kernel,
        grid=(_NB0 + _NQ1,),
        in_specs=[
            pl.BlockSpec((4, _BT0, _HIDDEN), lambda i: (0, jnp.minimum(i, _NB0 - 1), 0)),
            pl.BlockSpec((1, _HIDDEN), lambda i: (0, 0)),
        ],
        out_specs=[
            pl.BlockSpec((_BT0, _HIDDEN), lambda i: (jnp.minimum(i, _N_SPILL - 1), 0)),
            pl.BlockSpec((1, 128), lambda i: (0, jnp.minimum(i, _NB0 - 1))),
            pl.BlockSpec(
                (_BT1, _HIDDEN),
                lambda i: (jnp.where(i < _NB0, _RES_Q_BLOCK0, i - _NB0 + _RES_Q_BLOCK0), 0),
            ),
            pl.BlockSpec(memory_space=pltpu.SMEM),
        ],
        out_shape=[
            jax.ShapeDtypeStruct((_SPILL_TOKENS, _HIDDEN), jnp.bfloat16),
            jax.ShapeDtypeStruct((1, _NB0 * 128), jnp.float32),
            jax.ShapeDtypeStruct((_TOKENS, _HIDDEN), jnp.float32),
            jax.ShapeDtypeStruct((1, 1), jnp.float32),
        ],
        scratch_shapes=[
            pltpu.VMEM((_RES_TOKENS, _HIDDEN), jnp.bfloat16),
            pltpu.VMEM((_NB0, 128), jnp.float32),
        ],
        compiler_params=pltpu.CompilerParams(
            dimension_semantics=("arbitrary",),
            vmem_limit_bytes=56 * 1024 * 1024,
        ),
        name="allreduce_norm_resident",
    )(hidden_states, w2d)

    q = pl.pallas_call(
        _spill_quant_kernel,
        grid=(_NB2,),
        in_specs=[
            pl.BlockSpec((1, _NB0 * 128), lambda i: (0, 0)),
            pl.BlockSpec((_BT2, _HIDDEN), lambda i: (i, 0)),
            pl.BlockSpec(memory_space=pl.ANY),
        ],
        out_specs=pl.BlockSpec((_BT2, _HIDDEN), lambda i: (i, 0)),
        out_shape=jax.ShapeDtypeStruct((_TOKENS, _HIDDEN), jnp.float32),
        input_output_aliases={2: 0},
        compiler_params=pltpu.CompilerParams(
            dimension_semantics=("arbitrary",),
        ),
        name="spill_quant",
    )(pamax, spill, q_partial)

    return q, scale.reshape(())
